# Initial kernel scaffold; baseline (speedup 1.0000x reference)
import jax
import jax.numpy as jnp
from jax.experimental import pallas as pl


def kernel(x):
    raise NotImplementedError("write your pallas kernel here")


# speedup vs baseline: 2.1729x; 2.1729x over previous
import jax
import jax.numpy as jnp
import numpy as np
from jax import lax
from jax.experimental import pallas as pl
from jax.experimental.pallas import tpu as pltpu
from jax.experimental.pallas import tpu_sc as plsc

ROWS = 128
N = 32768
L = 16
SEG = N // L
PSTRIDE = SEG + 1
PADN = L * PSTRIDE
NBINS = 256 * L
MIN32 = np.int32(-2147483648)


def _sort_kernel(x_hbm, out_hbm, raw_v, a_v, b_v, hist_v):
  info = plsc.get_sparse_core_info()
  nc, ns = info.num_cores, info.num_subcores
  nw = nc * ns
  rpw = ROWS // nw

  iota = lax.iota(jnp.int32, L)
  seg_iota = iota * PSTRIDE
  ones = jnp.full((L,), 1, jnp.int32)
  zeros = jnp.zeros((L,), jnp.int32)

  wid = lax.axis_index("s") * nc + lax.axis_index("c")

  def flip_pass(i, _):
    off = pl.multiple_of(i * L, L)
    v = raw_v[pl.ds(off, L)]
    k = plsc.bitcast(v, jnp.int32)
    m = (k >> 31) | MIN32
    y = k ^ m
    base = i * L + (i >> 7)
    plsc.store_scatter(a_v, [base + iota], y)
    return 0

  def make_hist(in_ref, sh):
    def zbody(j, _):
      hist_v[pl.ds(pl.multiple_of(j * L, L), L)] = zeros
      return 0
    lax.fori_loop(0, NBINS // L, zbody, 0)

    def hbody(i, _):
      k = plsc.load_gather(in_ref, [seg_iota + i])
      d = (k >> sh) & 0xFF
      plsc.addupdate_scatter(hist_v, [(d << 4) + iota], ones)
      return 0
    lax.fori_loop(0, SEG, hbody, 0)

  def prefix_scan():
    def sbody(d, running):
      off = pl.multiple_of(d * L, L)
      v = hist_v[pl.ds(off, L)]
      cs = plsc.cumsum(v)
      total = jnp.sum(v)
      hist_v[pl.ds(off, L)] = cs - v + running
      return running + total
    lax.fori_loop(0, NBINS // L, sbody, jnp.int32(0))

  def rank_pass(in_ref, out_ref, sh):
    def rbody(i, _):
      k = plsc.load_gather(in_ref, [seg_iota + i])
      d = (k >> sh) & 0xFF
      bins = (d << 4) + iota
      pos = plsc.load_gather(hist_v, [bins])
      plsc.addupdate_scatter(hist_v, [bins], ones)
      plsc.store_scatter(out_ref, [pos + (pos >> 11)], k)
      return 0
    lax.fori_loop(0, SEG, rbody, 0)

  def final_pass(in_ref, sh):
    def fbody(i, _):
      k = plsc.load_gather(in_ref, [seg_iota + i])
      d = (k >> sh) & 0xFF
      bins = (d << 4) + iota
      pos = plsc.load_gather(hist_v, [bins])
      plsc.addupdate_scatter(hist_v, [bins], ones)
      m = (~k >> 31) | MIN32
      plsc.store_scatter(raw_v, [pos], plsc.bitcast(k ^ m, jnp.float32))
      return 0
    lax.fori_loop(0, SEG, fbody, 0)

  for r in range(rpw):
    row = wid * rpw + r
    pltpu.sync_copy(x_hbm.at[row], raw_v)
    lax.fori_loop(0, SEG, flip_pass, 0)
    for pss, (src, dst) in enumerate(((a_v, b_v), (b_v, a_v), (a_v, b_v))):
      make_hist(src, 8 * pss)
      prefix_scan()
      rank_pass(src, dst, 8 * pss)
    make_hist(b_v, 24)
    prefix_scan()
    final_pass(b_v, 24)
    pltpu.sync_copy(raw_v, out_hbm.at[row])


def kernel(x):
  mesh = plsc.VectorSubcoreMesh(core_axis_name="c", subcore_axis_name="s")
  f = pl.kernel(
      _sort_kernel,
      out_type=jax.ShapeDtypeStruct((ROWS, N), jnp.float32),
      mesh=mesh,
      compiler_params=pltpu.CompilerParams(needs_layout_passes=False),
      scratch_types=[
          pltpu.VMEM((N,), jnp.float32),
          pltpu.VMEM((PADN,), jnp.int32),
          pltpu.VMEM((PADN,), jnp.int32),
          pltpu.VMEM((NBINS,), jnp.int32),
      ],
  )
  return f(x)


# speedup vs baseline: 2.2469x; 1.0341x over previous
import jax
import jax.numpy as jnp
import numpy as np
from jax import lax
from jax.experimental import pallas as pl
from jax.experimental.pallas import tpu as pltpu
from jax.experimental.pallas import tpu_sc as plsc

ROWS = 128
N = 32768
L = 16
SEG = N // L
PSTRIDE = SEG + 1
PADN = L * PSTRIDE
NBINS = 256 * L
NHIST = 4
UN = 4
MIN32 = np.int32(-2147483648)


def _sort_kernel(x_hbm, out_hbm, raw_v, a_v, b_v, hist_v):
  info = plsc.get_sparse_core_info()
  nc, ns = info.num_cores, info.num_subcores
  nw = nc * ns
  rpw = ROWS // nw

  iota = lax.iota(jnp.int32, L)
  seg_iota = iota * PSTRIDE
  ones = jnp.full((L,), 1, jnp.int32)
  zeros = jnp.zeros((L,), jnp.int32)

  wid = lax.axis_index("s") * nc + lax.axis_index("c")

  def flip(k):
    return k ^ ((k >> 31) | MIN32)

  def zero_hists():
    def zbody(j, _):
      for u in range(8):
        hist_v[pl.ds(pl.multiple_of((j * 8 + u) * L, L), L)] = zeros
      return 0
    lax.fori_loop(0, NHIST * NBINS // L // 8, zbody, 0)

  def hist_pass(load_digit):
    zero_hists()

    def hbody(j, _):
      for u in range(UN):
        i = j * UN + u
        d = load_digit(i)
        plsc.addupdate_scatter(
            hist_v, [(d << 4) + iota + (u % NHIST) * NBINS], ones)
      return 0
    lax.fori_loop(0, SEG // UN, hbody, 0)

  def prefix_scan():
    def sbody(d, running):
      off = pl.multiple_of(d * L, L)
      v = hist_v[pl.ds(off, L)]
      for h in range(1, NHIST):
        v = v + hist_v[pl.ds(off + h * NBINS, L)]
      cs = plsc.cumsum(v)
      hist_v[pl.ds(off, L)] = cs - v + running
      return running + jnp.sum(v)
    lax.fori_loop(0, NBINS // L, sbody, jnp.int32(0))

  def rank_pass(load_key, store_fn):
    def rbody(i, _):
      y = load_key(i)
      bins = (((y >> (store_fn.sh)) & 0xFF) << 4) + iota
      pos = plsc.load_gather(hist_v, [bins])
      plsc.addupdate_scatter(hist_v, [bins], ones)
      store_fn(pos, y)
      return 0
    lax.fori_loop(0, SEG, rbody, 0)

  def raw_digit(i):
    off = pl.multiple_of(i * L, L)
    return flip(plsc.bitcast(raw_v[pl.ds(off, L)], jnp.int32)) & 0xFF

  def raw_key(i):
    off = pl.multiple_of(i * L, L)
    return flip(plsc.bitcast(raw_v[pl.ds(off, L)], jnp.int32))

  def make_padded_digit(src, sh):
    def g(i):
      return (plsc.load_gather(src, [seg_iota + i]) >> sh) & 0xFF
    return g

  def make_padded_key(src):
    def g(i):
      return plsc.load_gather(src, [seg_iota + i])
    return g

  def make_padded_store(dst, sh):
    def s(pos, y):
      plsc.store_scatter(dst, [pos + (pos >> 11)], y)
    s.sh = sh
    return s

  def final_store(pos, y):
    m = (~y >> 31) | MIN32
    plsc.store_scatter(raw_v, [pos], plsc.bitcast(y ^ m, jnp.float32))
  final_store.sh = 24

  for r in range(rpw):
    row = wid * rpw + r
    pltpu.sync_copy(x_hbm.at[row], raw_v)
    hist_pass(raw_digit)
    prefix_scan()
    rank_pass(raw_key, make_padded_store(a_v, 0))
    for pss, (src, dst) in enumerate(((a_v, b_v), (b_v, a_v)), start=1):
      hist_pass(make_padded_digit(src, 8 * pss))
      prefix_scan()
      rank_pass(make_padded_key(src), make_padded_store(dst, 8 * pss))
    hist_pass(make_padded_digit(a_v, 24))
    prefix_scan()
    rank_pass(make_padded_key(a_v), final_store)
    pltpu.sync_copy(raw_v, out_hbm.at[row])


def kernel(x):
  mesh = plsc.VectorSubcoreMesh(core_axis_name="c", subcore_axis_name="s")
  f = pl.kernel(
      _sort_kernel,
      out_type=jax.ShapeDtypeStruct((ROWS, N), jnp.float32),
      mesh=mesh,
      compiler_params=pltpu.CompilerParams(needs_layout_passes=False),
      scratch_types=[
          pltpu.VMEM((N,), jnp.float32),
          pltpu.VMEM((PADN,), jnp.int32),
          pltpu.VMEM((PADN,), jnp.int32),
          pltpu.VMEM((NHIST * NBINS,), jnp.int32),
      ],
  )
  return f(x)


# speedup vs baseline: 2.3763x; 1.0576x over previous
import jax
import jax.numpy as jnp
import numpy as np
from jax import lax
from jax.experimental import pallas as pl
from jax.experimental.pallas import tpu as pltpu
from jax.experimental.pallas import tpu_sc as plsc

ROWS = 128
N = 32768
L = 16
SEG = N // L
PSTRIDE = SEG + 1
PADN = L * PSTRIDE
NBINS = 256 * L
NQ = 4
QSEG = SEG // NQ
MIN32 = np.int32(-2147483648)


def _sort_kernel(x_hbm, out_hbm, raw_v, a_v, b_v, *hists):
  info = plsc.get_sparse_core_info()
  nc, ns = info.num_cores, info.num_subcores
  nw = nc * ns
  rpw = ROWS // nw

  iota = lax.iota(jnp.int32, L)
  seg_iota = iota * PSTRIDE
  ones = jnp.full((L,), 1, jnp.int32)
  zeros = jnp.zeros((L,), jnp.int32)

  wid = lax.axis_index("s") * nc + lax.axis_index("c")

  def flip(k):
    return k ^ ((k >> 31) | MIN32)

  def hist_pass(load_digit):
    def zbody(j, _):
      off = pl.multiple_of(j * L, L)
      for h in range(NQ):
        hists[h][pl.ds(off, L)] = zeros
      return 0
    lax.fori_loop(0, NBINS // L, zbody, 0)

    def hbody(j, _):
      for h in range(NQ):
        d = load_digit(j + h * QSEG)
        plsc.addupdate_scatter(hists[h], [(d << 4) + iota], ones)
      return 0
    lax.fori_loop(0, QSEG, hbody, 0)

  def prefix_scan():
    def sbody(d, running):
      off = pl.multiple_of(d * L, L)
      vs = [hists[h][pl.ds(off, L)] for h in range(NQ)]
      total = vs[0]
      for h in range(1, NQ):
        total = total + vs[h]
      cs = plsc.cumsum(total)
      g = cs - total + running
      for h in range(NQ):
        hists[h][pl.ds(off, L)] = g
        if h + 1 < NQ:
          g = g + vs[h]
      return running + jnp.sum(total)
    lax.fori_loop(0, NBINS // L, sbody, jnp.int32(0))

  def rank_pass(load_key, store_fn, sh):
    def rbody(j, _):
      for h in range(NQ):
        y = load_key(j + h * QSEG)
        bins = (((y >> sh) & 0xFF) << 4) + iota
        pos = plsc.load_gather(hists[h], [bins])
        plsc.addupdate_scatter(hists[h], [bins], ones)
        store_fn(pos, y)
      return 0
    lax.fori_loop(0, QSEG, rbody, 0)

  def raw_digit(sh):
    def g(i):
      off = pl.multiple_of(i * L, L)
      d = flip(plsc.bitcast(raw_v[pl.ds(off, L)], jnp.int32))
      return (d >> sh) & 0xFF
    return g

  def raw_key(i):
    off = pl.multiple_of(i * L, L)
    return flip(plsc.bitcast(raw_v[pl.ds(off, L)], jnp.int32))

  def padded_digit(src, sh):
    def g(i):
      d = plsc.load_gather(src, [seg_iota + i])
      return (d >> sh) & 0xFF
    return g

  def padded_key(src):
    def g(i):
      return plsc.load_gather(src, [seg_iota + i])
    return g

  def padded_store(dst):
    def s(pos, y):
      plsc.store_scatter(dst, [pos + (pos >> 11)], y)
    return s

  def final_store(pos, y):
    m = (~y >> 31) | MIN32
    plsc.store_scatter(raw_v, [pos], plsc.bitcast(y ^ m, jnp.float32))

  for r in range(rpw):
    row = wid * rpw + r
    pltpu.sync_copy(x_hbm.at[row], raw_v)
    hist_pass(raw_digit(0))
    prefix_scan()
    rank_pass(raw_key, padded_store(a_v), 0)
    for pss, (src, dst) in enumerate(((a_v, b_v), (b_v, a_v)), start=1):
      hist_pass(padded_digit(src, 8 * pss))
      prefix_scan()
      rank_pass(padded_key(src), padded_store(dst), 8 * pss)
    hist_pass(padded_digit(a_v, 24))
    prefix_scan()
    rank_pass(padded_key(a_v), final_store, 24)
    pltpu.sync_copy(raw_v, out_hbm.at[row])


def kernel(x):
  mesh = plsc.VectorSubcoreMesh(core_axis_name="c", subcore_axis_name="s")
  f = pl.kernel(
      _sort_kernel,
      out_type=jax.ShapeDtypeStruct((ROWS, N), jnp.float32),
      mesh=mesh,
      compiler_params=pltpu.CompilerParams(needs_layout_passes=False),
      scratch_types=[
          pltpu.VMEM((N,), jnp.float32),
          pltpu.VMEM((PADN,), jnp.int32),
          pltpu.VMEM((PADN,), jnp.int32),
      ] + [pltpu.VMEM((NBINS,), jnp.int32) for _ in range(NQ)],
  )
  return f(x)


# speedup vs baseline: 6.1892x; 2.6046x over previous
import jax
import jax.numpy as jnp
import numpy as np
from jax import lax
from jax.experimental import pallas as pl
from jax.experimental.pallas import tpu as pltpu
from jax.experimental.pallas import tpu_sc as plsc

ROWS = 128
N = 32768
L = 16
SEG = N // L
PSTRIDE = SEG + 1
PADN = L * PSTRIDE
NBINS = 256 * L
NQ = 4
QSEG = SEG // NQ
MIN32 = np.int32(-2147483648)


def _sort_kernel(x_hbm, out_hbm, raw_v, a_v, b_v, *hists):
  info = plsc.get_sparse_core_info()
  nc, ns = info.num_cores, info.num_subcores
  nw = nc * ns
  rpw = ROWS // nw

  iota = lax.iota(jnp.int32, L)
  seg_iota = iota * PSTRIDE
  ones = jnp.full((L,), 1, jnp.int32)
  zeros = jnp.zeros((L,), jnp.int32)

  wid = lax.axis_index("s") * nc + lax.axis_index("c")

  def flip(k):
    return k ^ ((k >> 31) | MIN32)

  def hist_pass(load_digit):
    def zbody(j, _):
      off = pl.multiple_of(j * L, L)
      for h in range(NQ):
        hists[h][pl.ds(off, L)] = zeros
      return 0
    lax.fori_loop(0, NBINS // L, zbody, 0)

    def hbody(j, _):
      ds = [load_digit(j + h * QSEG) for h in range(NQ)]
      bins = [(d << 4) + iota for d in ds]
      for h in range(NQ):
        plsc.addupdate_scatter(hists[h], [bins[h]], ones)
      return 0
    lax.fori_loop(0, QSEG, hbody, 0)

  def prefix_scan():
    def sbody(d, running):
      off = pl.multiple_of(d * L, L)
      vs = [hists[h][pl.ds(off, L)] for h in range(NQ)]
      total = vs[0]
      for h in range(1, NQ):
        total = total + vs[h]
      cs = plsc.cumsum(total)
      g = cs - total + running
      for h in range(NQ):
        hists[h][pl.ds(off, L)] = g
        if h + 1 < NQ:
          g = g + vs[h]
      return running + jnp.sum(total)
    lax.fori_loop(0, NBINS // L, sbody, jnp.int32(0))

  def rank_pass(load_key, store_fn, sh):
    def rbody(j, _):
      ys = [load_key(j + h * QSEG) for h in range(NQ)]
      bins = [(((y >> sh) & 0xFF) << 4) + iota for y in ys]
      poss = [plsc.load_gather(hists[h], [bins[h]]) for h in range(NQ)]
      for h in range(NQ):
        plsc.addupdate_scatter(hists[h], [bins[h]], ones)
      for h in range(NQ):
        store_fn(poss[h], ys[h])
      return 0
    lax.fori_loop(0, QSEG, rbody, 0)

  def raw_digit(sh):
    def g(i):
      off = pl.multiple_of(i * L, L)
      d = flip(plsc.bitcast(raw_v[pl.ds(off, L)], jnp.int32))
      return (d >> sh) & 0xFF
    return g

  def raw_key(i):
    off = pl.multiple_of(i * L, L)
    return flip(plsc.bitcast(raw_v[pl.ds(off, L)], jnp.int32))

  def padded_digit(src, sh):
    def g(i):
      d = plsc.load_gather(src, [seg_iota + i])
      return (d >> sh) & 0xFF
    return g

  def padded_key(src):
    def g(i):
      return plsc.load_gather(src, [seg_iota + i])
    return g

  def padded_store(dst):
    def s(pos, y):
      plsc.store_scatter(dst, [pos + (pos >> 11)], y)
    return s

  def final_store(pos, y):
    m = (~y >> 31) | MIN32
    plsc.store_scatter(raw_v, [pos], plsc.bitcast(y ^ m, jnp.float32))

  for r in range(rpw):
    row = wid * rpw + r
    pltpu.sync_copy(x_hbm.at[row], raw_v)
    hist_pass(raw_digit(0))
    prefix_scan()
    rank_pass(raw_key, padded_store(a_v), 0)
    for pss, (src, dst) in enumerate(((a_v, b_v), (b_v, a_v)), start=1):
      hist_pass(padded_digit(src, 8 * pss))
      prefix_scan()
      rank_pass(padded_key(src), padded_store(dst), 8 * pss)
    hist_pass(padded_digit(a_v, 24))
    prefix_scan()
    rank_pass(padded_key(a_v), final_store, 24)
    pltpu.sync_copy(raw_v, out_hbm.at[row])


def kernel(x):
  mesh = plsc.VectorSubcoreMesh(core_axis_name="c", subcore_axis_name="s")
  f = pl.kernel(
      _sort_kernel,
      out_type=jax.ShapeDtypeStruct((ROWS, N), jnp.float32),
      mesh=mesh,
      compiler_params=pltpu.CompilerParams(needs_layout_passes=False),
      scratch_types=[
          pltpu.VMEM((N,), jnp.float32),
          pltpu.VMEM((PADN,), jnp.int32),
          pltpu.VMEM((PADN,), jnp.int32),
      ] + [pltpu.VMEM((NBINS,), jnp.int32) for _ in range(NQ)],
  )
  return f(x)


# speedup vs baseline: 7.6167x; 1.2307x over previous
import jax
import jax.numpy as jnp
import numpy as np
from jax import lax
from jax.experimental import pallas as pl
from jax.experimental.pallas import tpu as pltpu
from jax.experimental.pallas import tpu_sc as plsc

ROWS = 128
N = 32768
L = 16
SEG = N // L
PSTRIDE = SEG + 1
PADN = L * PSTRIDE
NBINS = 256 * L
NQ = 8
QSEG = SEG // NQ
MIN32 = np.int32(-2147483648)


def _sort_kernel(x_hbm, out_hbm, buf0, buf1, sem_in, sem_out, *hists):
  info = plsc.get_sparse_core_info()
  nc, ns = info.num_cores, info.num_subcores
  nw = nc * ns
  rpw = ROWS // nw

  iota = lax.iota(jnp.int32, L)
  seg_iota = iota * PSTRIDE
  ones = jnp.full((L,), 1, jnp.int32)
  zeros = jnp.zeros((L,), jnp.int32)

  wid = lax.axis_index("s") * nc + lax.axis_index("c")

  def flip(k):
    return k ^ ((k >> 31) | MIN32)

  def hist_pass(load_digit):
    def zbody(j, _):
      off = pl.multiple_of(j * L, L)
      for h in range(NQ):
        hists[h][pl.ds(off, L)] = zeros
      return 0
    lax.fori_loop(0, NBINS // L, zbody, 0)

    def hbody(j, _):
      ds = [load_digit(j + h * QSEG) for h in range(NQ)]
      bins = [(d << 4) + iota for d in ds]
      for h in range(NQ):
        plsc.addupdate_scatter(hists[h], [bins[h]], ones)
      return 0
    lax.fori_loop(0, QSEG, hbody, 0)

  def prefix_scan():
    def sbody(d, running):
      off = pl.multiple_of(d * L, L)
      vs = [hists[h][pl.ds(off, L)] for h in range(NQ)]
      total = vs[0]
      for h in range(1, NQ):
        total = total + vs[h]
      cs = plsc.cumsum(total)
      g = cs - total + running
      for h in range(NQ):
        hists[h][pl.ds(off, L)] = g
        if h + 1 < NQ:
          g = g + vs[h]
      return running + jnp.sum(total)
    lax.fori_loop(0, NBINS // L, sbody, jnp.int32(0))

  def rank_pass(load_key, store_fn, sh):
    def rbody(j, _):
      ys = [load_key(j + h * QSEG) for h in range(NQ)]
      bins = [(((y >> sh) & 0xFF) << 4) + iota for y in ys]
      poss = [plsc.load_gather(hists[h], [bins[h]]) for h in range(NQ)]
      for h in range(NQ):
        plsc.addupdate_scatter(hists[h], [bins[h]], ones)
      for h in range(NQ):
        store_fn(poss[h], ys[h])
      return 0
    lax.fori_loop(0, QSEG, rbody, 0)

  def seq_digit(src, sh):
    def g(i):
      off = pl.multiple_of(i * L, L)
      d = flip(plsc.bitcast(src[pl.ds(off, L)], jnp.int32))
      return (d >> sh) & 0xFF
    return g

  def seq_key(src):
    def g(i):
      off = pl.multiple_of(i * L, L)
      return flip(plsc.bitcast(src[pl.ds(off, L)], jnp.int32))
    return g

  def padded_digit(src, sh):
    def g(i):
      d = plsc.bitcast(plsc.load_gather(src, [seg_iota + i]), jnp.int32)
      return (d >> sh) & 0xFF
    return g

  def padded_key(src):
    def g(i):
      return plsc.bitcast(plsc.load_gather(src, [seg_iota + i]), jnp.int32)
    return g

  def padded_store(dst):
    def s(pos, y):
      plsc.store_scatter(dst, [pos + (pos >> 11)],
                         plsc.bitcast(y, jnp.float32))
    return s

  def make_final_store(dst):
    def s(pos, y):
      m = (~y >> 31) | MIN32
      plsc.store_scatter(dst, [pos], plsc.bitcast(y ^ m, jnp.float32))
    return s

  bufs = (buf0, buf1)
  in_copy = pltpu.make_async_copy(
      x_hbm.at[wid * rpw], buf0.at[pl.ds(0, N)], sem_in)
  in_copy.start()
  out_copy = None
  for r in range(rpw):
    row = wid * rpw + r
    cin = bufs[r % 2]
    oth = bufs[1 - r % 2]
    in_copy.wait()
    hist_pass(seq_digit(cin, 0))
    prefix_scan()
    if out_copy is not None:
      out_copy.wait()
    rank_pass(seq_key(cin), padded_store(oth), 0)
    hist_pass(padded_digit(oth, 8))
    prefix_scan()
    rank_pass(padded_key(oth), padded_store(cin), 8)
    hist_pass(padded_digit(cin, 16))
    prefix_scan()
    rank_pass(padded_key(cin), padded_store(oth), 16)
    hist_pass(padded_digit(oth, 24))
    prefix_scan()
    rank_pass(padded_key(oth), make_final_store(cin), 24)
    out_copy = pltpu.make_async_copy(
        cin.at[pl.ds(0, N)], out_hbm.at[row], sem_out)
    out_copy.start()
    if r + 1 < rpw:
      in_copy = pltpu.make_async_copy(
          x_hbm.at[row + 1], oth.at[pl.ds(0, N)], sem_in)
      in_copy.start()
  out_copy.wait()


def kernel(x):
  mesh = plsc.VectorSubcoreMesh(core_axis_name="c", subcore_axis_name="s")
  f = pl.kernel(
      _sort_kernel,
      out_type=jax.ShapeDtypeStruct((ROWS, N), jnp.float32),
      mesh=mesh,
      compiler_params=pltpu.CompilerParams(needs_layout_passes=False),
      scratch_types=[
          pltpu.VMEM((PADN,), jnp.float32),
          pltpu.VMEM((PADN,), jnp.float32),
          pltpu.SemaphoreType.DMA,
          pltpu.SemaphoreType.DMA,
      ] + [pltpu.VMEM((NBINS,), jnp.int32) for _ in range(NQ)],
  )
  return f(x)
